# baseline (speedup 1.0000x reference)
import jax
import jax.numpy as jnp
from jax.experimental import pallas as pl
from jax.experimental.pallas import tpu as pltpu

_B = 16
_NPG = 1024
_D = 128
_K = 16
_KL = 127
_R = 256
_NBLK = _NPG // _R


def _mlp_body(x_ref, w1_ref, b1_ref, w2_ref, b2_ref, o_ref):
    h = jnp.maximum(jnp.dot(x_ref[...], w1_ref[...]) + b1_ref[...], 0.0)
    o_ref[...] = jnp.dot(h, w2_ref[...]) + b2_ref[...]


def _extract_topk(v_ref, pay_ref, idx_out_ref, pay_out_ref, n_iter):
    rows, width = v_ref.shape
    G = 8
    lane = jax.lax.broadcasted_iota(jnp.int32, (G, width), 1)
    slot = jax.lax.broadcasted_iota(jnp.int32, (G, 128), 1)
    big = jnp.int32(1 << 30)
    inf = jnp.float32(jnp.inf)
    has_pay = pay_ref is not None

    def group(g, carry):
        r0 = g * G
        vg = v_ref[pl.ds(r0, G), :]
        pg = pay_ref[pl.ds(r0, G), :] if has_pay else None

        def body(s, c):
            vv, idx_acc, pay_acc = c
            m = jnp.min(vv, axis=1, keepdims=True)
            jcand = jnp.where(vv == m, lane, big)
            j = jnp.min(jcand, axis=1, keepdims=True)
            sel = jcand == j
            if has_pay:
                pay = jnp.sum(jnp.where(sel, pg, 0.0), axis=1, keepdims=True)
                pay_acc = jnp.where(slot == s, pay, pay_acc)
            vv = jnp.where(sel, inf, vv)
            idx_acc = jnp.where(slot == s, j, idx_acc)
            return vv, idx_acc, pay_acc

        idx0 = jnp.zeros((G, 128), jnp.int32)
        pay0 = jnp.zeros((G, 128), jnp.float32)
        _, idx_acc, pay_acc = jax.lax.fori_loop(0, n_iter, body,
                                                (vg, idx0, pay0))
        idx_out_ref[pl.ds(r0, G), :] = idx_acc
        if has_pay:
            pay_out_ref[pl.ds(r0, G), :] = pay_acc
        return carry

    jax.lax.fori_loop(0, rows // G, group, 0)


def _graph_body(x_ref, xgT_ref, sqx_r_ref, sqx_c_ref,
                emb_ref, embT_ref, sqe_r_ref, sqe_c_ref,
                pos_ref, posT_ref, sqp_r_ref, sqp_c_ref,
                gum_ref, t_ref,
                srcx_ref, val_ref, srcp_ref,
                dv_scr, dp_scr, idx_scr, pay_scr):
    i = pl.program_id(0)
    rb = i % _NBLK
    t = t_ref[0, 0]
    dn = (((1,), (0,)), ((), ()))
    lane1024 = jax.lax.broadcasted_iota(jnp.int32, (_R, _NPG), 1)
    row_g = jax.lax.broadcasted_iota(jnp.int32, (_R, 1), 0) + rb * _R
    diag = lane1024 == row_g

    dotx = jax.lax.dot_general(x_ref[...], xgT_ref[0], dn,
                               preferred_element_type=jnp.float32)
    d2x = (sqx_r_ref[0] + sqx_c_ref[0]) - 2.0 * dotx
    dv_scr[...] = jnp.where(diag, d2x + 1e10, d2x)

    dote = jax.lax.dot_general(emb_ref[...], embT_ref[0], dn,
                               preferred_element_type=jnp.float32)
    dp_scr[...] = (sqe_r_ref[0] + sqe_c_ref[0]) - 2.0 * dote

    _extract_topk(dv_scr, dp_scr, idx_scr, pay_scr, _KL)
    idx127 = idx_scr[...]
    de2_127 = pay_scr[...]

    p = jnp.exp(-t * de2_127)
    noisy = jnp.log(p + 1e-20) + gum_ref[...]
    slot = jax.lax.broadcasted_iota(jnp.int32, (_R, 128), 1)
    noisy = jnp.where(slot < _KL, noisy, -jnp.inf)

    lane128 = slot
    big = jnp.int32(1 << 30)
    ninf = jnp.float32(-jnp.inf)

    def body2(s, carry):
        nv, v_acc, src_acc = carry
        m = jnp.max(nv, axis=1, keepdims=True)
        jcand = jnp.where(nv == m, lane128, big)
        j = jnp.min(jcand, axis=1, keepdims=True)
        sel = lane128 == j
        val = jnp.sum(jnp.where(sel, noisy, 0.0), axis=1, keepdims=True)
        src = jnp.sum(jnp.where(sel, idx127, 0), axis=1, keepdims=True)
        v_acc = jnp.where(slot == s, val, v_acc)
        src_acc = jnp.where(slot == s, src, src_acc)
        nv = jnp.where(sel, ninf, nv)
        return nv, v_acc, src_acc

    v0 = jnp.zeros((_R, 128), jnp.float32)
    s0 = jnp.zeros((_R, 128), jnp.int32)
    _, v_acc, src_acc = jax.lax.fori_loop(0, _K, body2, (noisy, v0, s0))

    topv = v_acc[:, :_K]
    mx = jnp.max(topv, axis=1, keepdims=True)
    e = jnp.exp(topv - mx)
    topv = e / jnp.sum(e, axis=1, keepdims=True)

    goff = (i // _NBLK) * _NPG
    srcx_ref[...] = src_acc[:, :_K] + goff
    val_ref[...] = topv

    dotp = jax.lax.dot_general(pos_ref[...], posT_ref[0], dn,
                               preferred_element_type=jnp.float32)
    d2p = (sqp_r_ref[0] + sqp_c_ref[0]) - 2.0 * dotp
    dv_scr[...] = jnp.where(diag, d2p + 1e10, d2p)
    _extract_topk(dv_scr, None, idx_scr, None, _K)
    srcp_ref[...] = idx_scr[:, :_K] + goff


def _make_call(interpret=False):
    nb = _B * _NBLK
    bg = lambda i: (i // _NBLK, 0, 0)
    br = lambda i: (i, 0)
    br3 = lambda i: (i, 0, 0)
    return pl.pallas_call(
        _graph_body,
        out_shape=(
            jax.ShapeDtypeStruct((_B * _NPG, _K), jnp.int32),
            jax.ShapeDtypeStruct((_B * _NPG, _K), jnp.float32),
            jax.ShapeDtypeStruct((_B * _NPG, _K), jnp.int32),
        ),
        grid=(nb,),
        in_specs=[
            pl.BlockSpec((_R, _D), br),
            pl.BlockSpec((1, _D, _NPG), bg),
            pl.BlockSpec((1, _R, 1), br3),
            pl.BlockSpec((1, 1, _NPG), bg),
            pl.BlockSpec((_R, 20), br),
            pl.BlockSpec((1, 20, _NPG), bg),
            pl.BlockSpec((1, _R, 1), br3),
            pl.BlockSpec((1, 1, _NPG), bg),
            pl.BlockSpec((_R, 3), br),
            pl.BlockSpec((1, 3, _NPG), bg),
            pl.BlockSpec((1, _R, 1), br3),
            pl.BlockSpec((1, 1, _NPG), bg),
            pl.BlockSpec((_R, 128), br),
            pl.BlockSpec((1, 1), lambda i: (0, 0)),
        ],
        out_specs=(
            pl.BlockSpec((_R, _K), br),
            pl.BlockSpec((_R, _K), br),
            pl.BlockSpec((_R, _K), br),
        ),
        scratch_shapes=[
            pltpu.VMEM((_R, _NPG), jnp.float32),
            pltpu.VMEM((_R, _NPG), jnp.float32),
            pltpu.VMEM((_R, 128), jnp.int32),
            pltpu.VMEM((_R, 128), jnp.float32),
        ],
        interpret=interpret,
    )


def kernel(x, pos, batch, W1, b1, W2, b2, t, interpret=False):
    n = _B * _NPG
    emb = pl.pallas_call(
        _mlp_body,
        out_shape=jax.ShapeDtypeStruct((n, 20), jnp.float32),
        grid=(8,),
        in_specs=[
            pl.BlockSpec((2048, _D), lambda i: (i, 0)),
            pl.BlockSpec((_D, _D), lambda i: (0, 0)),
            pl.BlockSpec((1, _D), lambda i: (0, 0)),
            pl.BlockSpec((_D, 20), lambda i: (0, 0)),
            pl.BlockSpec((1, 20), lambda i: (0, 0)),
        ],
        out_specs=pl.BlockSpec((2048, 20), lambda i: (i, 0)),
        interpret=interpret,
    )(x, W1, b1.reshape(1, _D), W2, b2.reshape(1, 20))

    kr = jax.random.key(1)
    rand_scores = jax.random.uniform(jax.random.fold_in(kr, 0), emb.shape,
                                     dtype=emb.dtype) * 1e-4
    emb = emb + rand_scores
    u = jax.random.uniform(jax.random.fold_in(kr, 1), (n, _KL), dtype=jnp.float32)
    gum = -jnp.log(-jnp.log(u + 1e-20) + 1e-20)
    gum = jnp.concatenate([gum, jnp.zeros((n, 1), jnp.float32)], axis=1)

    xg = x.reshape(_B, _NPG, _D)
    eg = emb.reshape(_B, _NPG, 20)
    pg = pos.reshape(_B, _NPG, 3)
    sqx = jnp.sum(xg * xg, axis=-1)
    sqe = jnp.sum(eg * eg, axis=-1)
    sqp = jnp.sum(pg * pg, axis=-1)

    srcx, topv, srcp = _make_call(interpret)(
        x, xg.transpose(0, 2, 1), sqx.reshape(_B * _NBLK, _R, 1),
        sqx.reshape(_B, 1, _NPG),
        emb, eg.transpose(0, 2, 1), sqe.reshape(_B * _NBLK, _R, 1),
        sqe.reshape(_B, 1, _NPG),
        pos, pg.transpose(0, 2, 1), sqp.reshape(_B * _NBLK, _R, 1),
        sqp.reshape(_B, 1, _NPG),
        gum, t.reshape(1, 1),
    )

    dst = jnp.repeat(jnp.arange(n, dtype=jnp.int32), _K)
    edges_sparse = jnp.stack([srcx.reshape(-1), dst])
    edge_index = jnp.stack([srcp.reshape(-1), dst])
    topv_f = topv.reshape(-1)
    edges_sparse_v = jnp.stack([topv_f, dst.astype(jnp.float32)], axis=0)
    edge_index_out = jnp.concatenate([edges_sparse, edge_index], axis=1)
    return (edge_index_out, edges_sparse, edges_sparse_v)


# speedup vs baseline: 13.2567x; 13.2567x over previous
import jax
import jax.numpy as jnp
from jax.experimental import pallas as pl
from jax.experimental.pallas import tpu as pltpu

_B = 16
_NPG = 1024
_D = 128
_K = 16
_KL = 127
_R = 256
_NBLK = _NPG // _R


def _mlp_body(x_ref, w1_ref, b1_ref, w2_ref, b2_ref, o_ref):
    h = jnp.maximum(jnp.dot(x_ref[...], w1_ref[...]) + b1_ref[...], 0.0)
    o_ref[...] = jnp.dot(h, w2_ref[...]) + b2_ref[...]


def _extract_topk(v_ref, pay_ref, idx_out_ref, pay_out_ref, n_iter):
    rows, width = v_ref.shape
    lane = jax.lax.broadcasted_iota(jnp.int32, (rows, width), 1)
    slot = jax.lax.broadcasted_iota(jnp.int32, (rows, 128), 1)
    big = jnp.int32(1 << 30)
    inf = jnp.float32(jnp.inf)
    has_pay = pay_ref is not None
    v = v_ref[...]
    pg = pay_ref[...] if has_pay else None

    def body(s, carry):
        vv, idx_acc, pay_acc = carry
        m = jnp.min(vv, axis=1, keepdims=True)
        jcand = jnp.where(vv == m, lane, big)
        j = jnp.min(jcand, axis=1, keepdims=True)
        sel = jcand == j
        if has_pay:
            pay = jnp.sum(jnp.where(sel, pg, 0.0), axis=1, keepdims=True)
            pay_acc = jnp.where(slot == s, pay, pay_acc)
        vv = jnp.where(sel, inf, vv)
        idx_acc = jnp.where(slot == s, j, idx_acc)
        return vv, idx_acc, pay_acc

    idx0 = jnp.zeros((rows, 128), jnp.int32)
    pay0 = jnp.zeros((rows, 128), jnp.float32)
    _, idx_acc, pay_acc = jax.lax.fori_loop(0, n_iter, body, (v, idx0, pay0))
    idx_out_ref[...] = idx_acc
    if has_pay:
        pay_out_ref[...] = pay_acc


def _graph_body(x_ref, xgT_ref, sqx_r_ref, sqx_c_ref,
                emb_ref, embT_ref, sqe_r_ref, sqe_c_ref,
                pos_ref, posT_ref, sqp_r_ref, sqp_c_ref,
                gum_ref, t_ref,
                srcx_ref, val_ref, srcp_ref,
                dv_scr, dp_scr, idx_scr, pay_scr):
    i = pl.program_id(0)
    rb = i % _NBLK
    t = t_ref[0, 0]
    dn = (((1,), (0,)), ((), ()))
    lane1024 = jax.lax.broadcasted_iota(jnp.int32, (_R, _NPG), 1)
    row_g = jax.lax.broadcasted_iota(jnp.int32, (_R, 1), 0) + rb * _R
    diag = lane1024 == row_g

    dotx = jax.lax.dot_general(x_ref[...], xgT_ref[0], dn,
                               preferred_element_type=jnp.float32)
    d2x = (sqx_r_ref[0] + sqx_c_ref[0]) - 2.0 * dotx
    dv_scr[...] = jnp.where(diag, d2x + 1e10, d2x)

    dote = jax.lax.dot_general(emb_ref[...], embT_ref[0], dn,
                               preferred_element_type=jnp.float32)
    dp_scr[...] = (sqe_r_ref[0] + sqe_c_ref[0]) - 2.0 * dote

    _extract_topk(dv_scr, dp_scr, idx_scr, pay_scr, _KL)
    idx127 = idx_scr[...]
    de2_127 = pay_scr[...]

    p = jnp.exp(-t * de2_127)
    noisy = jnp.log(p + 1e-20) + gum_ref[...]
    slot = jax.lax.broadcasted_iota(jnp.int32, (_R, 128), 1)
    noisy = jnp.where(slot < _KL, noisy, -jnp.inf)

    lane128 = slot
    big = jnp.int32(1 << 30)
    ninf = jnp.float32(-jnp.inf)

    def body2(s, carry):
        nv, v_acc, src_acc = carry
        m = jnp.max(nv, axis=1, keepdims=True)
        jcand = jnp.where(nv == m, lane128, big)
        j = jnp.min(jcand, axis=1, keepdims=True)
        sel = lane128 == j
        val = jnp.sum(jnp.where(sel, noisy, 0.0), axis=1, keepdims=True)
        src = jnp.sum(jnp.where(sel, idx127, 0), axis=1, keepdims=True)
        v_acc = jnp.where(slot == s, val, v_acc)
        src_acc = jnp.where(slot == s, src, src_acc)
        nv = jnp.where(sel, ninf, nv)
        return nv, v_acc, src_acc

    v0 = jnp.zeros((_R, 128), jnp.float32)
    s0 = jnp.zeros((_R, 128), jnp.int32)
    _, v_acc, src_acc = jax.lax.fori_loop(0, _K, body2, (noisy, v0, s0))

    topv = v_acc[:, :_K]
    mx = jnp.max(topv, axis=1, keepdims=True)
    e = jnp.exp(topv - mx)
    topv = e / jnp.sum(e, axis=1, keepdims=True)

    goff = (i // _NBLK) * _NPG
    srcx_ref[...] = src_acc[:, :_K] + goff
    val_ref[...] = topv

    dotp = jax.lax.dot_general(pos_ref[...], posT_ref[0], dn,
                               preferred_element_type=jnp.float32)
    d2p = (sqp_r_ref[0] + sqp_c_ref[0]) - 2.0 * dotp
    dv_scr[...] = jnp.where(diag, d2p + 1e10, d2p)
    _extract_topk(dv_scr, None, idx_scr, None, _K)
    srcp_ref[...] = idx_scr[:, :_K] + goff


def _make_call(interpret=False):
    nb = _B * _NBLK
    bg = lambda i: (i // _NBLK, 0, 0)
    br = lambda i: (i, 0)
    br3 = lambda i: (i, 0, 0)
    return pl.pallas_call(
        _graph_body,
        out_shape=(
            jax.ShapeDtypeStruct((_B * _NPG, _K), jnp.int32),
            jax.ShapeDtypeStruct((_B * _NPG, _K), jnp.float32),
            jax.ShapeDtypeStruct((_B * _NPG, _K), jnp.int32),
        ),
        grid=(nb,),
        in_specs=[
            pl.BlockSpec((_R, _D), br),
            pl.BlockSpec((1, _D, _NPG), bg),
            pl.BlockSpec((1, _R, 1), br3),
            pl.BlockSpec((1, 1, _NPG), bg),
            pl.BlockSpec((_R, 20), br),
            pl.BlockSpec((1, 20, _NPG), bg),
            pl.BlockSpec((1, _R, 1), br3),
            pl.BlockSpec((1, 1, _NPG), bg),
            pl.BlockSpec((_R, 3), br),
            pl.BlockSpec((1, 3, _NPG), bg),
            pl.BlockSpec((1, _R, 1), br3),
            pl.BlockSpec((1, 1, _NPG), bg),
            pl.BlockSpec((_R, 128), br),
            pl.BlockSpec((1, 1), lambda i: (0, 0)),
        ],
        out_specs=(
            pl.BlockSpec((_R, _K), br),
            pl.BlockSpec((_R, _K), br),
            pl.BlockSpec((_R, _K), br),
        ),
        scratch_shapes=[
            pltpu.VMEM((_R, _NPG), jnp.float32),
            pltpu.VMEM((_R, _NPG), jnp.float32),
            pltpu.VMEM((_R, 128), jnp.int32),
            pltpu.VMEM((_R, 128), jnp.float32),
        ],
        interpret=interpret,
    )


def kernel(x, pos, batch, W1, b1, W2, b2, t, interpret=False):
    n = _B * _NPG
    emb = pl.pallas_call(
        _mlp_body,
        out_shape=jax.ShapeDtypeStruct((n, 20), jnp.float32),
        grid=(8,),
        in_specs=[
            pl.BlockSpec((2048, _D), lambda i: (i, 0)),
            pl.BlockSpec((_D, _D), lambda i: (0, 0)),
            pl.BlockSpec((1, _D), lambda i: (0, 0)),
            pl.BlockSpec((_D, 20), lambda i: (0, 0)),
            pl.BlockSpec((1, 20), lambda i: (0, 0)),
        ],
        out_specs=pl.BlockSpec((2048, 20), lambda i: (i, 0)),
        interpret=interpret,
    )(x, W1, b1.reshape(1, _D), W2, b2.reshape(1, 20))

    kr = jax.random.key(1)
    rand_scores = jax.random.uniform(jax.random.fold_in(kr, 0), emb.shape,
                                     dtype=emb.dtype) * 1e-4
    emb = emb + rand_scores
    u = jax.random.uniform(jax.random.fold_in(kr, 1), (n, _KL), dtype=jnp.float32)
    gum = -jnp.log(-jnp.log(u + 1e-20) + 1e-20)
    gum = jnp.concatenate([gum, jnp.zeros((n, 1), jnp.float32)], axis=1)

    xg = x.reshape(_B, _NPG, _D)
    eg = emb.reshape(_B, _NPG, 20)
    pg = pos.reshape(_B, _NPG, 3)
    sqx = jnp.sum(xg * xg, axis=-1)
    sqe = jnp.sum(eg * eg, axis=-1)
    sqp = jnp.sum(pg * pg, axis=-1)

    srcx, topv, srcp = _make_call(interpret)(
        x, xg.transpose(0, 2, 1), sqx.reshape(_B * _NBLK, _R, 1),
        sqx.reshape(_B, 1, _NPG),
        emb, eg.transpose(0, 2, 1), sqe.reshape(_B * _NBLK, _R, 1),
        sqe.reshape(_B, 1, _NPG),
        pos, pg.transpose(0, 2, 1), sqp.reshape(_B * _NBLK, _R, 1),
        sqp.reshape(_B, 1, _NPG),
        gum, t.reshape(1, 1),
    )

    dst = jnp.repeat(jnp.arange(n, dtype=jnp.int32), _K)
    edges_sparse = jnp.stack([srcx.reshape(-1), dst])
    edge_index = jnp.stack([srcp.reshape(-1), dst])
    topv_f = topv.reshape(-1)
    edges_sparse_v = jnp.stack([topv_f, dst.astype(jnp.float32)], axis=0)
    edge_index_out = jnp.concatenate([edges_sparse, edge_index], axis=1)
    return (edge_index_out, edges_sparse, edges_sparse_v)


# speedup vs baseline: 13.8527x; 1.0450x over previous
import jax
import jax.numpy as jnp
from jax.experimental import pallas as pl
from jax.experimental.pallas import tpu as pltpu

_B = 16
_NPG = 1024
_D = 128
_K = 16
_KL = 127
_R = 256
_NBLK = _NPG // _R


def _mlp_body(x_ref, w1_ref, b1_ref, w2_ref, b2_ref, o_ref):
    h = jnp.maximum(jnp.dot(x_ref[...], w1_ref[...]) + b1_ref[...], 0.0)
    o_ref[...] = jnp.dot(h, w2_ref[...]) + b2_ref[...]


def _extract_topk(v, payload, n_iter):
    rows, width = v.shape
    lane = jax.lax.broadcasted_iota(jnp.int32, (rows, width), 1)
    slot = jax.lax.broadcasted_iota(jnp.int32, (rows, 128), 1)
    big = jnp.int32(1 << 30)
    inf = jnp.float32(jnp.inf)
    has_pay = payload is not None

    def body(s, carry):
        vv, idx_acc, pay_acc = carry
        m = jnp.min(vv, axis=1, keepdims=True)
        jcand = jnp.where(vv == m, lane, big)
        j = jnp.min(jcand, axis=1, keepdims=True)
        sel = jcand == j
        if has_pay:
            pay = jnp.sum(jnp.where(sel, payload, 0.0), axis=1, keepdims=True)
            pay_acc = jnp.where(slot == s, pay, pay_acc)
        vv = jnp.where(sel, inf, vv)
        idx_acc = jnp.where(slot == s, j, idx_acc)
        return vv, idx_acc, pay_acc

    idx0 = jnp.zeros((rows, 128), jnp.int32)
    pay0 = jnp.zeros((rows, 128), jnp.float32)
    _, idx_acc, pay_acc = jax.lax.fori_loop(0, n_iter, body, (v, idx0, pay0))
    return idx_acc, pay_acc


def _graph_body(x_ref, xgT_ref, sqx_r_ref, sqx_c_ref,
                emb_ref, embT_ref, sqe_r_ref, sqe_c_ref,
                pos_ref, posT_ref, sqp_r_ref, sqp_c_ref,
                gum_ref, t_ref,
                srcx_ref, val_ref, srcp_ref):
    i = pl.program_id(0)
    rb = i % _NBLK
    t = t_ref[0, 0]
    dn = (((1,), (0,)), ((), ()))
    lane1024 = jax.lax.broadcasted_iota(jnp.int32, (_R, _NPG), 1)
    row_g = jax.lax.broadcasted_iota(jnp.int32, (_R, 1), 0) + rb * _R
    diag = lane1024 == row_g

    dotx = jax.lax.dot_general(x_ref[...], xgT_ref[0], dn,
                               preferred_element_type=jnp.float32)
    d2x = (sqx_r_ref[0] + sqx_c_ref[0]) - 2.0 * dotx
    d2x = jnp.where(diag, d2x + 1e10, d2x)

    embT = embT_ref[0]
    emb_blk = emb_ref[...]
    de2 = jnp.zeros((_R, _NPG), jnp.float32)
    for d in range(20):
        diff = emb_blk[:, d:d + 1] - embT[d:d + 1, :]
        de2 = de2 + diff * diff

    idx127, de2_127 = _extract_topk(d2x, de2, _KL)

    dist = jnp.sqrt(de2_127)
    p = jnp.exp(-t * (dist * dist))
    noisy = jnp.log(p + 1e-20) + gum_ref[...]
    slot = jax.lax.broadcasted_iota(jnp.int32, (_R, 128), 1)
    noisy = jnp.where(slot < _KL, noisy, -jnp.inf)

    lane128 = slot
    big = jnp.int32(1 << 30)
    ninf = jnp.float32(-jnp.inf)

    def body2(s, carry):
        nv, v_acc, src_acc = carry
        m = jnp.max(nv, axis=1, keepdims=True)
        jcand = jnp.where(nv == m, lane128, big)
        j = jnp.min(jcand, axis=1, keepdims=True)
        sel = lane128 == j
        val = jnp.sum(jnp.where(sel, noisy, 0.0), axis=1, keepdims=True)
        src = jnp.sum(jnp.where(sel, idx127, 0), axis=1, keepdims=True)
        v_acc = jnp.where(slot == s, val, v_acc)
        src_acc = jnp.where(slot == s, src, src_acc)
        nv = jnp.where(sel, ninf, nv)
        return nv, v_acc, src_acc

    v0 = jnp.zeros((_R, 128), jnp.float32)
    s0 = jnp.zeros((_R, 128), jnp.int32)
    _, v_acc, src_acc = jax.lax.fori_loop(0, _K, body2, (noisy, v0, s0))

    topv = v_acc[:, :_K]
    mx = jnp.max(topv, axis=1, keepdims=True)
    e = jnp.exp(topv - mx)
    topv = e / jnp.sum(e, axis=1, keepdims=True)

    goff = (i // _NBLK) * _NPG
    srcx_ref[...] = src_acc[:, :_K] + goff
    val_ref[...] = topv

    dotp = jax.lax.dot_general(pos_ref[...], posT_ref[0], dn,
                               preferred_element_type=jnp.float32)
    d2p = (sqp_r_ref[0] + sqp_c_ref[0]) - 2.0 * dotp
    d2p = jnp.where(diag, d2p + 1e10, d2p)
    idxp, _ = _extract_topk(d2p, None, _K)
    srcp_ref[...] = idxp[:, :_K] + goff


def _make_call(interpret=False):
    nb = _B * _NBLK
    bg = lambda i: (i // _NBLK, 0, 0)
    br = lambda i: (i, 0)
    br3 = lambda i: (i, 0, 0)
    return pl.pallas_call(
        _graph_body,
        out_shape=(
            jax.ShapeDtypeStruct((_B * _NPG, _K), jnp.int32),
            jax.ShapeDtypeStruct((_B * _NPG, _K), jnp.float32),
            jax.ShapeDtypeStruct((_B * _NPG, _K), jnp.int32),
        ),
        grid=(nb,),
        in_specs=[
            pl.BlockSpec((_R, _D), br),
            pl.BlockSpec((1, _D, _NPG), bg),
            pl.BlockSpec((1, _R, 1), br3),
            pl.BlockSpec((1, 1, _NPG), bg),
            pl.BlockSpec((_R, 20), br),
            pl.BlockSpec((1, 20, _NPG), bg),
            pl.BlockSpec((1, _R, 1), br3),
            pl.BlockSpec((1, 1, _NPG), bg),
            pl.BlockSpec((_R, 3), br),
            pl.BlockSpec((1, 3, _NPG), bg),
            pl.BlockSpec((1, _R, 1), br3),
            pl.BlockSpec((1, 1, _NPG), bg),
            pl.BlockSpec((_R, 128), br),
            pl.BlockSpec((1, 1), lambda i: (0, 0)),
        ],
        out_specs=(
            pl.BlockSpec((_R, _K), br),
            pl.BlockSpec((_R, _K), br),
            pl.BlockSpec((_R, _K), br),
        ),
        interpret=interpret,
    )


def kernel(x, pos, batch, W1, b1, W2, b2, t, interpret=False):
    n = _B * _NPG
    emb = pl.pallas_call(
        _mlp_body,
        out_shape=jax.ShapeDtypeStruct((n, 20), jnp.float32),
        grid=(8,),
        in_specs=[
            pl.BlockSpec((2048, _D), lambda i: (i, 0)),
            pl.BlockSpec((_D, _D), lambda i: (0, 0)),
            pl.BlockSpec((1, _D), lambda i: (0, 0)),
            pl.BlockSpec((_D, 20), lambda i: (0, 0)),
            pl.BlockSpec((1, 20), lambda i: (0, 0)),
        ],
        out_specs=pl.BlockSpec((2048, 20), lambda i: (i, 0)),
        interpret=interpret,
    )(x, W1, b1.reshape(1, _D), W2, b2.reshape(1, 20))

    kr = jax.random.key(1)
    rand_scores = jax.random.uniform(jax.random.fold_in(kr, 0), emb.shape,
                                     dtype=emb.dtype) * 1e-4
    emb = emb + rand_scores
    u = jax.random.uniform(jax.random.fold_in(kr, 1), (n, _KL), dtype=jnp.float32)
    gum = -jnp.log(-jnp.log(u + 1e-20) + 1e-20)
    gum = jnp.concatenate([gum, jnp.zeros((n, 1), jnp.float32)], axis=1)

    xg = x.reshape(_B, _NPG, _D)
    eg = emb.reshape(_B, _NPG, 20)
    pg = pos.reshape(_B, _NPG, 3)
    sqx = jnp.sum(xg * xg, axis=-1)
    sqe = jnp.sum(eg * eg, axis=-1)
    sqp = jnp.sum(pg * pg, axis=-1)

    srcx, topv, srcp = _make_call(interpret)(
        x, xg.transpose(0, 2, 1), sqx.reshape(_B * _NBLK, _R, 1),
        sqx.reshape(_B, 1, _NPG),
        emb, eg.transpose(0, 2, 1), sqe.reshape(_B * _NBLK, _R, 1),
        sqe.reshape(_B, 1, _NPG),
        pos, pg.transpose(0, 2, 1), sqp.reshape(_B * _NBLK, _R, 1),
        sqp.reshape(_B, 1, _NPG),
        gum, t.reshape(1, 1),
    )

    dst = jnp.repeat(jnp.arange(n, dtype=jnp.int32), _K)
    edges_sparse = jnp.stack([srcx.reshape(-1), dst])
    edge_index = jnp.stack([srcp.reshape(-1), dst])
    topv_f = topv.reshape(-1)
    edges_sparse_v = jnp.stack([topv_f, dst.astype(jnp.float32)], axis=0)
    edge_index_out = jnp.concatenate([edges_sparse, edge_index], axis=1)
    return (edge_index_out, edges_sparse, edges_sparse_v)
